# baseline (speedup 1.0000x reference)
import jax
import jax.numpy as jnp
from jax.experimental import pallas as pl

_N, _D, _P = 4096, 256, 2
_BM = 512
_NB = _N // _BM
_BM2 = 1024
_NB2 = _N // _BM2


def _s_body(h_ref, wt_ref, s_ref):
    s_ref[0] = jnp.dot(h_ref[...], wt_ref[0],
                       preferred_element_type=jnp.float32)


def _main_body(adj_ref, s_ref, b_ref, a_ref, awt_ref, ab_ref, e_ref, stat_ref):
    i = pl.program_id(1)
    o = jnp.dot(adj_ref[0], s_ref[0], preferred_element_type=jnp.float32)
    o = o + b_ref[0]
    e = jnp.maximum(o, 0.0) + a_ref[0] * jnp.minimum(o, 0.0)
    e_ref[0] = e
    t = jnp.tanh(jnp.dot(e, awt_ref[...],
                         preferred_element_type=jnp.float32) + ab_ref[...])

    @pl.when(i == 0)
    def _init():
        stat_ref[...] = jnp.zeros_like(stat_ref)

    stat_ref[0] = stat_ref[0] + jnp.sum(t, axis=0, keepdims=True)


def _comb_body(stat_ref, att_ref, e_ref, z_ref):
    l0 = jnp.sum(att_ref[...] * stat_ref[0]) * (1.0 / _N)
    l1 = jnp.sum(att_ref[...] * stat_ref[1]) * (1.0 / _N)
    m = jnp.maximum(l0, l1)
    w0 = jnp.exp(l0 - m)
    w1 = jnp.exp(l1 - m)
    beta0 = w0 / (w0 + w1)
    beta1 = w1 / (w0 + w1)
    z_ref[...] = beta0 * e_ref[0] + beta1 * e_ref[1]


def kernel(h, mps, W0, b0, a0, W1, b1, a1, attW, attb, att):
    wt = jnp.stack([W0.T, W1.T])

    s = pl.pallas_call(
        _s_body,
        grid=(_P,),
        in_specs=[pl.BlockSpec((_N, _D), lambda p: (0, 0)),
                  pl.BlockSpec((1, _D, _D), lambda p: (p, 0, 0))],
        out_specs=pl.BlockSpec((1, _N, _D), lambda p: (p, 0, 0)),
        out_shape=jax.ShapeDtypeStruct((_P, _N, _D), jnp.float32),
    )(h, wt)

    bias = jnp.stack([b0, b1]).reshape(_P, 1, _D)
    aa = jnp.broadcast_to(jnp.stack([a0, a1])[:, None, None], (_P, 1, _D))
    awt = attW.T
    ab = attb[None, :]

    e, stat = pl.pallas_call(
        _main_body,
        grid=(_P, _NB),
        in_specs=[
            pl.BlockSpec((1, _BM, _N), lambda p, i: (p, i, 0)),
            pl.BlockSpec((1, _N, _D), lambda p, i: (p, 0, 0)),
            pl.BlockSpec((1, 1, _D), lambda p, i: (p, 0, 0)),
            pl.BlockSpec((1, 1, _D), lambda p, i: (p, 0, 0)),
            pl.BlockSpec((_D, _D), lambda p, i: (0, 0)),
            pl.BlockSpec((1, _D), lambda p, i: (0, 0)),
        ],
        out_specs=[
            pl.BlockSpec((1, _BM, _D), lambda p, i: (p, i, 0)),
            pl.BlockSpec((1, 1, _D), lambda p, i: (p, 0, 0)),
        ],
        out_shape=[jax.ShapeDtypeStruct((_P, _N, _D), jnp.float32),
                   jax.ShapeDtypeStruct((_P, 1, _D), jnp.float32)],
    )(mps, s, bias, aa, awt, ab)

    z = pl.pallas_call(
        _comb_body,
        grid=(_NB2,),
        in_specs=[
            pl.BlockSpec((_P, 1, _D), lambda j: (0, 0, 0)),
            pl.BlockSpec((1, _D), lambda j: (0, 0)),
            pl.BlockSpec((_P, _BM2, _D), lambda j: (0, j, 0)),
        ],
        out_specs=pl.BlockSpec((_BM2, _D), lambda j: (j, 0)),
        out_shape=jax.ShapeDtypeStruct((_N, _D), jnp.float32),
    )(stat, att, e)
    return z


# speedup vs baseline: 1.1213x; 1.1213x over previous
import jax
import jax.numpy as jnp
from jax.experimental import pallas as pl
from jax.experimental.pallas import tpu as pltpu

_N, _D, _P = 4096, 256, 2
_BM = 512
_NB = _N // _BM
_BM2 = 1024
_NB2 = _N // _BM2


def _main_body(h_ref, wt_ref, adj_ref, b_ref, a_ref, awt_ref, ab_ref,
               e_ref, stat_ref, s_ref):
    i = pl.program_id(1)

    @pl.when(i == 0)
    def _compute_s():
        s_ref[...] = jnp.dot(h_ref[...], wt_ref[0],
                             preferred_element_type=jnp.float32)

    o = jnp.dot(adj_ref[0], s_ref[...], preferred_element_type=jnp.float32)
    o = o + b_ref[0]
    e = jnp.maximum(o, 0.0) + a_ref[0] * jnp.minimum(o, 0.0)
    e_ref[0] = e.astype(jnp.bfloat16)
    t = jnp.tanh(jnp.dot(e, awt_ref[...],
                         preferred_element_type=jnp.float32) + ab_ref[...])

    @pl.when(i == 0)
    def _init():
        stat_ref[...] = jnp.zeros_like(stat_ref)

    stat_ref[0] = stat_ref[0] + jnp.sum(t, axis=0, keepdims=True)


def _comb_body(stat_ref, att_ref, e_ref, z_ref):
    l0 = jnp.sum(att_ref[...] * stat_ref[0]) * (1.0 / _N)
    l1 = jnp.sum(att_ref[...] * stat_ref[1]) * (1.0 / _N)
    m = jnp.maximum(l0, l1)
    w0 = jnp.exp(l0 - m)
    w1 = jnp.exp(l1 - m)
    beta0 = w0 / (w0 + w1)
    beta1 = w1 / (w0 + w1)
    z_ref[...] = (beta0 * e_ref[0].astype(jnp.float32)
                  + beta1 * e_ref[1].astype(jnp.float32))


def kernel(h, mps, W0, b0, a0, W1, b1, a1, attW, attb, att):
    wt = jnp.stack([W0.T, W1.T])
    bias = jnp.stack([b0, b1]).reshape(_P, 1, _D)
    aa = jnp.broadcast_to(jnp.stack([a0, a1])[:, None, None], (_P, 1, _D))
    awt = attW.T
    ab = attb[None, :]

    e, stat = pl.pallas_call(
        _main_body,
        grid=(_P, _NB),
        in_specs=[
            pl.BlockSpec((_N, _D), lambda p, i: (0, 0)),
            pl.BlockSpec((1, _D, _D), lambda p, i: (p, 0, 0)),
            pl.BlockSpec((1, _BM, _N), lambda p, i: (p, i, 0)),
            pl.BlockSpec((1, 1, _D), lambda p, i: (p, 0, 0)),
            pl.BlockSpec((1, 1, _D), lambda p, i: (p, 0, 0)),
            pl.BlockSpec((_D, _D), lambda p, i: (0, 0)),
            pl.BlockSpec((1, _D), lambda p, i: (0, 0)),
        ],
        out_specs=[
            pl.BlockSpec((1, _BM, _D), lambda p, i: (p, i, 0)),
            pl.BlockSpec((1, 1, _D), lambda p, i: (p, 0, 0)),
        ],
        out_shape=[jax.ShapeDtypeStruct((_P, _N, _D), jnp.bfloat16),
                   jax.ShapeDtypeStruct((_P, 1, _D), jnp.float32)],
        scratch_shapes=[pltpu.VMEM((_N, _D), jnp.float32)],
    )(h, wt, mps, bias, aa, awt, ab)

    z = pl.pallas_call(
        _comb_body,
        grid=(_NB2,),
        in_specs=[
            pl.BlockSpec((_P, 1, _D), lambda j: (0, 0, 0)),
            pl.BlockSpec((1, _D), lambda j: (0, 0)),
            pl.BlockSpec((_P, _BM2, _D), lambda j: (0, j, 0)),
        ],
        out_specs=pl.BlockSpec((_BM2, _D), lambda j: (j, 0)),
        out_shape=jax.ShapeDtypeStruct((_N, _D), jnp.float32),
    )(stat, att, e)
    return z
